# baseline (speedup 1.0000x reference)
import functools

import jax
import jax.numpy as jnp
from jax.experimental import pallas as pl
from jax.experimental.pallas import tpu as pltpu
from jax.experimental.pallas import tpu_sc as plsc

BT = 1024
ET = 128


def _route_body(x_ref, wf_ref, bf_ref, wr_ref, br_ref,
                feat_ref, pos2_ref, t2e_ref,
                tri_ref, run_ref, comb_ref):
    bt = x_ref.shape[0]
    n_leaf = br_ref.shape[1]
    n_cls = feat_ref.shape[1]
    n_tok = comb_ref.shape[0]
    n_tiles = t2e_ref.shape[0]
    n_blocks = n_tok // bt
    i = pl.program_id(0)

    @pl.when(i == 0)
    def _():
        row = jax.lax.broadcasted_iota(jnp.int32, (bt, bt), 0)
        col = jax.lax.broadcasted_iota(jnp.int32, (bt, bt), 1)
        tri_ref[...] = (col < row).astype(jnp.float32)
        run_ref[...] = jnp.zeros_like(run_ref)

    @pl.when(i < n_blocks)
    def _():
        x = x_ref[...]
        feat_ref[...] = jnp.maximum(
            jnp.dot(x, wf_ref[...], preferred_element_type=jnp.float32)
            + bf_ref[...],
            0.0,
        )
        logits = (jnp.dot(x, wr_ref[...], preferred_element_type=jnp.float32)
                  + br_ref[...])
        lmax = jnp.max(logits, axis=1, keepdims=True)
        eidx = jax.lax.broadcasted_iota(jnp.int32, (bt, n_leaf), 1)
        choices = jnp.min(jnp.where(logits == lmax, eidx, n_leaf),
                          axis=1, keepdims=True)
        onehot = (eidx == choices).astype(jnp.float32)

        ranks = jnp.dot(tri_ref[...], onehot, preferred_element_type=jnp.float32)
        lpos = jnp.sum(onehot * (ranks + run_ref[...]), axis=1, keepdims=True)

        lane = jax.lax.broadcasted_iota(jnp.int32, (bt, 2 * n_leaf), 1)
        comb = (jnp.where(lane == choices, 1.0, 0.0)
                + jnp.where(lane == n_leaf, lpos, 0.0))
        comb_ref[pl.ds(i * bt, bt), :] = comb

        run_ref[...] = run_ref[...] + jnp.dot(
            jnp.ones((1, bt), jnp.float32), onehot,
            preferred_element_type=jnp.float32)

    @pl.when(i == n_blocks)
    def _():
        seg = (((run_ref[...].astype(jnp.int32) + (ET - 1)) >> 7) << 7)
        er = jax.lax.broadcasted_iota(jnp.int32, (n_leaf, 2 * n_leaf), 0)
        ec = jax.lax.broadcasted_iota(jnp.int32, (n_leaf, 2 * n_leaf), 1)
        m = ((er < ec) & (ec < n_leaf)).astype(jnp.float32)
        off = jnp.dot(seg.astype(jnp.float32), m,
                      preferred_element_type=jnp.float32)
        lane1 = jax.lax.broadcasted_iota(jnp.int32, (1, 2 * n_leaf), 1)
        wide = off + jnp.where(lane1 == n_leaf, 1.0, 0.0)
        ones_col = jnp.ones((2 * n_leaf, 1), jnp.float32)
        pos2_ref[...] = jnp.dot(comb_ref[...] * wide, ones_col,
                                preferred_element_type=jnp.float32
                                ).astype(jnp.int32)
        tstart = (jax.lax.broadcasted_iota(jnp.int32, (n_tiles, n_leaf), 0)
                  * ET).astype(jnp.float32)
        hit = (off[:, :n_leaf] <= tstart).astype(jnp.float32)
        t2e_ref[...] = jnp.dot(hit, ones_col[:n_leaf],
                               preferred_element_type=jnp.float32
                               ).astype(jnp.int32) - 1


def _sc_row_scatter(feat, pos2, n_pad):
    info = plsc.get_sparse_core_info()
    nc, ns = info.num_cores, info.num_subcores
    nw = nc * ns
    n_tok, d_f = feat.shape
    b_per_w = n_tok // nw
    mesh = plsc.VectorSubcoreMesh(core_axis_name="c", subcore_axis_name="s")

    @functools.partial(
        pl.kernel, mesh=mesh,
        out_type=jax.ShapeDtypeStruct((n_pad, d_f), jnp.float32),
        scratch_types=[
            pltpu.VMEM((b_per_w,), jnp.int32),
            pltpu.VMEM((b_per_w, d_f), jnp.float32),
            pltpu.SemaphoreType.DMA,
            pltpu.SemaphoreType.DMA,
        ],
    )
    def k(feat_hbm, idx_hbm, out_hbm, idx_v, rows_v, sem1, sem2):
        wid = jax.lax.axis_index("s") * nc + jax.lax.axis_index("c")
        base = wid * b_per_w
        cp_i = pltpu.async_copy(idx_hbm.at[pl.ds(base, b_per_w)], idx_v, sem1)
        cp_r = pltpu.async_copy(feat_hbm.at[pl.ds(base, b_per_w)], rows_v, sem2)
        cp_i.wait()
        cp_r.wait()
        pltpu.async_copy(rows_v, out_hbm.at[idx_v], sem1).wait()

    return k(feat, pos2)


def _expert_body(t2e_ref, featS_ref, w2_ref, lb_ref, out_ref):
    d_f = featS_ref.shape[1]
    n_leaf = lb_ref.shape[0]
    n_sub = featS_ref.shape[0] // ET
    t = pl.program_id(0)
    for s in range(n_sub):
        e = t2e_ref[t * n_sub + s, 0]
        w_e = w2_ref[pl.ds(e * d_f, d_f), :]
        acc = jnp.dot(featS_ref[pl.ds(s * ET, ET), :], w_e,
                      preferred_element_type=jnp.float32)
        eoh = (jax.lax.broadcasted_iota(jnp.int32, (1, n_leaf), 1) == e
               ).astype(jnp.float32)
        out_ref[pl.ds(s * ET, ET), :] = acc + jnp.dot(
            eoh, lb_ref[...], preferred_element_type=jnp.float32)


def kernel(inputs, Wf, bf, Wr, br, leaf_W, leaf_b):
    n_tok, d_in = inputs.shape
    d_f = Wf.shape[1]
    n_leaf, _, n_cls = leaf_W.shape
    n_pad = n_tok + n_leaf * ET
    n_tiles = n_pad // ET
    n_blocks = n_tok // BT

    feat, pos2_2d, t2e = pl.pallas_call(
        _route_body,
        grid=(n_blocks + 1,),
        in_specs=[
            pl.BlockSpec((BT, d_in), lambda i: (jnp.minimum(i, n_tok // BT - 1), 0)),
            pl.BlockSpec((d_in, d_f), lambda i: (0, 0)),
            pl.BlockSpec((1, d_f), lambda i: (0, 0)),
            pl.BlockSpec((d_in, n_leaf), lambda i: (0, 0)),
            pl.BlockSpec((1, n_leaf), lambda i: (0, 0)),
        ],
        out_specs=[
            pl.BlockSpec((BT, d_f), lambda i: (jnp.minimum(i, n_tok // BT - 1), 0)),
            pl.BlockSpec((n_tok, 1), lambda i: (0, 0)),
            pl.BlockSpec((n_tiles, 1), lambda i: (0, 0)),
        ],
        out_shape=[
            jax.ShapeDtypeStruct((n_tok, d_f), jnp.float32),
            jax.ShapeDtypeStruct((n_tok, 1), jnp.int32),
            jax.ShapeDtypeStruct((n_tiles, 1), jnp.int32),
        ],
        scratch_shapes=[
            pltpu.VMEM((BT, BT), jnp.float32),
            pltpu.VMEM((1, n_leaf), jnp.float32),
            pltpu.VMEM((n_tok, 2 * n_leaf), jnp.float32),
        ],
    )(inputs, Wf, bf.reshape(1, d_f), Wr, br.reshape(1, n_leaf))

    pos2 = pos2_2d[:, 0]
    featS = _sc_row_scatter(feat, pos2, n_pad)

    n_sub = 16
    predS = pl.pallas_call(
        _expert_body,
        grid=(n_tiles // n_sub,),
        in_specs=[
            pl.BlockSpec(memory_space=pltpu.SMEM),
            pl.BlockSpec((n_sub * ET, d_f), lambda t: (t, 0)),
            pl.BlockSpec((n_leaf * d_f, n_cls), lambda t: (0, 0)),
            pl.BlockSpec((n_leaf, n_cls), lambda t: (0, 0)),
        ],
        out_specs=pl.BlockSpec((n_sub * ET, n_cls), lambda t: (t, 0)),
        out_shape=jax.ShapeDtypeStruct((n_pad, n_cls), jnp.float32),
    )(t2e, featS, leaf_W.reshape(n_leaf * d_f, n_cls), leaf_b)

    return predS[pos2]


# speedup vs baseline: 1.3945x; 1.3945x over previous
import jax
import jax.numpy as jnp
from jax.experimental import pallas as pl

BT = 512


def _body(x_ref, wf_ref, bf_ref, wr_ref, br_ref, w2_ref, lb_ref, out_ref):
    bt = x_ref.shape[0]
    n_leaf, n_cls = lb_ref.shape
    d_f = wf_ref.shape[1]

    x = x_ref[...]
    feat = jnp.maximum(
        jnp.dot(x, wf_ref[...], preferred_element_type=jnp.float32) + bf_ref[...],
        0.0,
    )
    logits = jnp.dot(x, wr_ref[...], preferred_element_type=jnp.float32) + br_ref[...]
    lmax = jnp.max(logits, axis=1, keepdims=True)
    eidx = jax.lax.broadcasted_iota(jnp.int32, (bt, n_leaf), 1)
    choices = jnp.min(jnp.where(logits == lmax, eidx, n_leaf), axis=1, keepdims=True)
    onehot = (eidx == choices).astype(jnp.float32)

    acc = jnp.dot(onehot, lb_ref[...], preferred_element_type=jnp.float32)
    zero = jnp.zeros_like(feat)
    for e in range(n_leaf):
        mfe = jnp.where(choices == e, feat, zero)
        pe = jnp.dot(mfe, w2_ref[pl.ds(e * d_f, d_f), :],
                     preferred_element_type=jnp.float32)
        acc = acc + pe
    out_ref[...] = acc


def kernel(inputs, Wf, bf, Wr, br, leaf_W, leaf_b):
    n_tok, d_in = inputs.shape
    d_f = Wf.shape[1]
    n_leaf, _, n_cls = leaf_W.shape
    w2 = leaf_W.reshape(n_leaf * d_f, n_cls)
    grid = (n_tok // BT,)
    return pl.pallas_call(
        _body,
        grid=grid,
        in_specs=[
            pl.BlockSpec((BT, d_in), lambda i: (i, 0)),
            pl.BlockSpec((d_in, d_f), lambda i: (0, 0)),
            pl.BlockSpec((1, d_f), lambda i: (0, 0)),
            pl.BlockSpec((d_in, n_leaf), lambda i: (0, 0)),
            pl.BlockSpec((1, n_leaf), lambda i: (0, 0)),
            pl.BlockSpec((n_leaf * d_f, n_cls), lambda i: (0, 0)),
            pl.BlockSpec((n_leaf, n_cls), lambda i: (0, 0)),
        ],
        out_specs=pl.BlockSpec((BT, n_cls), lambda i: (i, 0)),
        out_shape=jax.ShapeDtypeStruct((n_tok, n_cls), jnp.float32),
    )(inputs, Wf, bf.reshape(1, d_f), Wr, br.reshape(1, n_leaf), w2, leaf_b)


# speedup vs baseline: 1.6360x; 1.1731x over previous
import jax
import jax.numpy as jnp
from jax.experimental import pallas as pl

BT = 512


def _body(x_ref, wf_ref, bf_ref, wr_ref, br_ref, w2_ref, lb_ref, out_ref):
    bt = x_ref.shape[0]
    n_leaf, n_cls = lb_ref.shape
    d_f = wf_ref.shape[1]

    x = x_ref[...]
    feat = jnp.maximum(
        jnp.dot(x, wf_ref[...], preferred_element_type=jnp.float32) + bf_ref[...],
        0.0,
    )
    logits = jnp.dot(x, wr_ref[...], preferred_element_type=jnp.float32) + br_ref[...]
    lmax = jnp.max(logits, axis=1, keepdims=True)
    eidx = jax.lax.broadcasted_iota(jnp.int32, (bt, n_leaf), 1)
    choices = jnp.min(jnp.where(logits == lmax, eidx, n_leaf), axis=1, keepdims=True)
    onehot = (eidx == choices).astype(jnp.float32)

    acc = jnp.dot(onehot, lb_ref[...], preferred_element_type=jnp.float32)
    for e in range(n_leaf):
        pe = jnp.dot(feat, w2_ref[pl.ds(e * d_f, d_f), :],
                     preferred_element_type=jnp.float32)
        acc = acc + jnp.where(choices == e, pe, 0.0)
    out_ref[...] = acc


def kernel(inputs, Wf, bf, Wr, br, leaf_W, leaf_b):
    n_tok, d_in = inputs.shape
    d_f = Wf.shape[1]
    n_leaf, _, n_cls = leaf_W.shape
    w2 = leaf_W.reshape(n_leaf * d_f, n_cls)
    grid = (n_tok // BT,)
    return pl.pallas_call(
        _body,
        grid=grid,
        in_specs=[
            pl.BlockSpec((BT, d_in), lambda i: (i, 0)),
            pl.BlockSpec((d_in, d_f), lambda i: (0, 0)),
            pl.BlockSpec((1, d_f), lambda i: (0, 0)),
            pl.BlockSpec((d_in, n_leaf), lambda i: (0, 0)),
            pl.BlockSpec((1, n_leaf), lambda i: (0, 0)),
            pl.BlockSpec((n_leaf * d_f, n_cls), lambda i: (0, 0)),
            pl.BlockSpec((n_leaf, n_cls), lambda i: (0, 0)),
        ],
        out_specs=pl.BlockSpec((BT, n_cls), lambda i: (i, 0)),
        out_shape=jax.ShapeDtypeStruct((n_tok, n_cls), jnp.float32),
    )(inputs, Wf, bf.reshape(1, d_f), Wr, br.reshape(1, n_leaf), w2, leaf_b)
